# Initial kernel scaffold; baseline (speedup 1.0000x reference)
import jax
import jax.numpy as jnp
from jax.experimental import pallas as pl


def kernel(nodes_u, nodes_pos, nodes_neg, hist_u, hist_ur, hist_v, hist_vr, soc_adj, params):
    raise NotImplementedError("write your pallas kernel here")


# speedup vs baseline: 1.8691x; 1.8691x over previous
import functools

import jax
import jax.numpy as jnp
from jax import lax
from jax.experimental import pallas as pl
from jax.experimental.pallas import tpu as pltpu
from jax.experimental.pallas import tpu_sc as plsc

NUSERS = 100000
NITEMS = 100000
D = 16
B = 4096
LH = 50
LS = 20

NC = 2
NS = 16
NW = NC * NS
BPW = B // NW
SUB = 64
NSUB = BPW // SUB


def _gather_rows(tab, rows_v, dst, sem, nrows, lrow):
  GR = 8

  def step(g, carry):
    descs = [pltpu.async_copy(tab.at[rows_v.at[g * GR + k]],
                              dst.at[pl.ds((g * GR + k) * lrow, lrow)], sem)
             for k in range(GR)]
    for d in descs:
      d.wait()
    return carry

  lax.fori_loop(0, nrows // GR, step, 0)


def _sc_gather_body(nodes_u, nodes_p, nodes_n, hist_u, hist_ur, hist_v,
                    hist_vr, soc, u2e, v2e,
                    eh_u, eh_p, eh_n, se_o, r_u, r_p, r_n, rp_u, rp_p, rp_n,
                    nodes_v, rows_v, rrows_v, srows_v, e_v, se_v, rep_v,
                    sem1, sem2, sem3):
  wid = lax.axis_index("s") * NC + lax.axis_index("c")

  def branch(base, nodes_hbm, hrow_hbm, rrow_hbm, rep_tab, emb_tab,
             eh_out, r_out, rep_out, do_soc):
    sl = pl.ds(base, SUB)
    pltpu.sync_copy(nodes_hbm.at[sl], nodes_v)
    d1 = pltpu.async_copy(hrow_hbm.at[nodes_v], rows_v, sem1)
    d2 = pltpu.async_copy(rrow_hbm.at[nodes_v], rrows_v, sem2)
    d3 = pltpu.async_copy(rep_tab.at[nodes_v], rep_v, sem3)
    if do_soc:
      d5 = pltpu.async_copy(soc.at[nodes_v], srows_v, sem3)
    d1.wait()
    _gather_rows(emb_tab, rows_v, e_v, sem1, SUB, LH)
    d2.wait()
    pltpu.sync_copy(rrows_v, r_out.at[sl])
    d3.wait()
    pltpu.sync_copy(rep_v, rep_out.at[sl])
    pltpu.sync_copy(e_v, eh_out.at[pl.ds(base * LH, SUB * LH)])
    if do_soc:
      d5.wait()
      _gather_rows(u2e, srows_v, se_v, sem3, SUB, LS)
      pltpu.sync_copy(se_v, se_o.at[pl.ds(base * LS, SUB * LS)])

  for s in range(NSUB):
    base = wid * BPW + s * SUB
    branch(base, nodes_u, hist_u, hist_ur, u2e, v2e, eh_u, r_u, rp_u, True)
    branch(base, nodes_p, hist_v, hist_vr, v2e, u2e, eh_p, r_p, rp_p, False)
    branch(base, nodes_n, hist_v, hist_vr, v2e, u2e, eh_n, r_n, rp_n, False)


def _sc_gather(nodes_u, nodes_p, nodes_n, hist_u, hist_ur, hist_v, hist_vr,
               soc, u2e, v2e):
  f32, i32 = jnp.float32, jnp.int32
  out_type = [
      jax.ShapeDtypeStruct((B * LH, D), f32),
      jax.ShapeDtypeStruct((B * LH, D), f32),
      jax.ShapeDtypeStruct((B * LH, D), f32),
      jax.ShapeDtypeStruct((B * LS, D), f32),
      jax.ShapeDtypeStruct((B, LH), i32),
      jax.ShapeDtypeStruct((B, LH), i32),
      jax.ShapeDtypeStruct((B, LH), i32),
      jax.ShapeDtypeStruct((B, D), f32),
      jax.ShapeDtypeStruct((B, D), f32),
      jax.ShapeDtypeStruct((B, D), f32),
  ]
  scratch = [
      pltpu.VMEM((SUB,), i32),
      pltpu.VMEM((SUB, LH), i32),
      pltpu.VMEM((SUB, LH), i32),
      pltpu.VMEM((SUB, LS), i32),
      pltpu.VMEM((SUB * LH, D), f32),
      pltpu.VMEM((SUB * LS, D), f32),
      pltpu.VMEM((SUB, D), f32),
      pltpu.SemaphoreType.DMA,
      pltpu.SemaphoreType.DMA,
      pltpu.SemaphoreType.DMA,
  ]
  fn = pl.kernel(
      _sc_gather_body,
      out_type=out_type,
      scratch_types=scratch,
      mesh=plsc.VectorSubcoreMesh(core_axis_name="c", subcore_axis_name="s"),
      compiler_params=pltpu.CompilerParams(use_tc_tiling_on_sc=False),
  )
  return fn(nodes_u, nodes_p, nodes_n, hist_u, hist_ur, hist_v, hist_vr,
            soc, u2e, v2e)


C = 64
G = B // C

_relu = lambda x: jnp.maximum(x, 0.0)


def _dot(a, b):
  return lax.dot_general(a, b, (((1,), (0,)), ((), ())),
                         preferred_element_type=jnp.float32)


def _neigh_agg(ehf, r, rep, p):
  m = C * LH
  oh = (r[:, :, None] ==
        lax.broadcasted_iota(jnp.int32, (C, LH, 8), 2)).astype(jnp.float32)
  h = _relu(_dot(ehf, p["W1a"]) + _dot(oh.reshape(m, 8), p["tab8"]) + p["b1"])
  o = _relu(_dot(h, p["W2"]) + p["b2"])
  rep_a = _dot(rep, p["A1b"]) + p["a1b"]
  x1 = _relu((_dot(o, p["A1a"]).reshape(C, LH, D) +
              rep_a[:, None, :]).reshape(m, D))
  x2 = _relu(_dot(x1, p["A2"]) + p["a2b"])
  s = jnp.sum(x2 * p["a3"], axis=1).reshape(C, LH)
  s = s - jnp.max(s, axis=1, keepdims=True)
  e = jnp.exp(s)
  att = e / jnp.sum(e, axis=1, keepdims=True)
  return jnp.sum(o.reshape(C, LH, D) * att[:, :, None], axis=1)


def _soc_agg(ef, rep, p):
  m = C * LS
  rep_a = _dot(rep, p["S1b"]) + p["s1b"]
  x1 = _relu((_dot(ef, p["S1a"]).reshape(C, LS, D) +
              rep_a[:, None, :]).reshape(m, D))
  x2 = _relu(_dot(x1, p["S2"]) + p["s2b"])
  s = jnp.sum(x2 * p["s3"], axis=1).reshape(C, LS)
  s = s - jnp.max(s, axis=1, keepdims=True)
  e = jnp.exp(s)
  att = e / jnp.sum(e, axis=1, keepdims=True)
  return jnp.sum(ef.reshape(C, LS, D) * att[:, :, None], axis=1)


def _tc1_body(eh_u, r_u, rep_u, soce, eh_p, r_p, rep_p, eh_n, r_n, rep_n,
              pp, xu_o, xi_o, xj_o):
  p = jax.tree.map(lambda r: r[...], pp)
  eh_u, r_u, rep_u, soce = eh_u[...], r_u[...], rep_u[...], soce[...]
  eh_p, r_p, rep_p = eh_p[...], r_p[...], rep_p[...]
  eh_n, r_n, rep_n = eh_n[...], r_n[...], rep_n[...]

  nu = _neigh_agg(eh_u, r_u, rep_u, p["u"])
  self_u = _relu(_dot(rep_u, p["EuhA"]) + _dot(nu, p["EuhB"]) + p["euhb"])
  ns = _soc_agg(soce, rep_u, p)
  emb_u = _relu(_dot(self_u, p["EuA"]) + _dot(ns, p["EuB"]) + p["eub"])
  xu_o[...] = _dot(emb_u, p["Wur1"]) + p["bur1"]

  np_ = _neigh_agg(eh_p, r_p, rep_p, p["v"])
  emb_i = _relu(_dot(rep_p, p["EvhA"]) + _dot(np_, p["EvhB"]) + p["evhb"])
  xi_o[...] = _dot(emb_i, p["Wvr1"]) + p["bvr1"]

  nn = _neigh_agg(eh_n, r_n, rep_n, p["v"])
  emb_j = _relu(_dot(rep_n, p["EvhA"]) + _dot(nn, p["EvhB"]) + p["evhb"])
  xj_o[...] = _dot(emb_j, p["Wvr1"]) + p["bvr1"]


def _tc2_body(xu, xi, xj, pp, out):
  p = jax.tree.map(lambda r: r[...], pp)

  def bn_head(x, g, b, w, bo):
    mean = jnp.mean(x, axis=0, keepdims=True)
    var = jnp.mean((x - mean) ** 2, axis=0, keepdims=True)
    xn = g * (x - mean) / jnp.sqrt(var + 1e-5) + b
    return _dot(_relu(xn), w) + bo

  x_u = bn_head(xu[...], p["g1"], p["b1"], p["Wur2"], p["bur2"])
  x_i = bn_head(xi[...], p["g2"], p["b2"], p["Wvr2"], p["bvr2"])
  x_j = bn_head(xj[...], p["g2"], p["b2"], p["Wvr2"], p["bvr2"])
  d = jnp.sum(x_u * x_i - x_u * x_j, axis=1)
  lp = jnp.sum(jnp.minimum(d, 0.0) - jnp.log(1.0 + jnp.exp(-jnp.abs(d))))
  reg = 1e-4 * (jnp.sum(x_u ** 2) + jnp.sum(x_i ** 2) + jnp.sum(x_j ** 2))
  out[...] = jnp.reshape(reg - lp, (1, 1))


def _prep_params(P):
  def split2(w):
    return w[:D], w[D:]

  pr = {}
  for tag, agg in (("u", P["agg_u"]), ("v", P["agg_v"])):
    w1a, w1b = split2(agg["w_r1_w"])
    tab8 = jnp.concatenate(
        [P["r2e"] @ w1b, jnp.zeros((3, D), jnp.float32)], axis=0)
    att = agg["att"]
    a1a, a1b_w = split2(att["a1w"])
    pr[tag] = dict(W1a=w1a, tab8=tab8, b1=agg["w_r1_b"][None],
                   W2=agg["w_r2_w"], b2=agg["w_r2_b"][None],
                   A1a=a1a, A1b=a1b_w, a1b=att["a1b"][None],
                   A2=att["a2w"], a2b=att["a2b"][None], a3=att["a3w"].T)
  s1a, s1b_w = split2(P["soc_att"]["a1w"])
  pr["S1a"], pr["S1b"] = s1a, s1b_w
  pr["s1b"] = P["soc_att"]["a1b"][None]
  pr["S2"], pr["s2b"] = P["soc_att"]["a2w"], P["soc_att"]["a2b"][None]
  pr["s3"] = P["soc_att"]["a3w"].T
  pr["EuhA"], pr["EuhB"] = split2(P["enc_uh_w"])
  pr["euhb"] = P["enc_uh_b"][None]
  pr["EvhA"], pr["EvhB"] = split2(P["enc_vh_w"])
  pr["evhb"] = P["enc_vh_b"][None]
  pr["EuA"], pr["EuB"] = split2(P["enc_u_w"])
  pr["eub"] = P["enc_u_b"][None]
  pr["Wur1"], pr["bur1"] = P["w_ur1_w"], P["w_ur1_b"][None]
  pr["Wvr1"], pr["bvr1"] = P["w_vr1_w"], P["w_vr1_b"][None]
  p2 = dict(Wur2=P["w_ur2_w"], bur2=P["w_ur2_b"][None],
            Wvr2=P["w_vr2_w"], bvr2=P["w_vr2_b"][None],
            g1=P["bn1_g"][None], b1=P["bn1_b"][None],
            g2=P["bn2_g"][None], b2=P["bn2_b"][None])
  return pr, p2


def _full(x):
  return pl.BlockSpec(x.shape, lambda *_: (0,) * x.ndim)


def _tc_stage1(eh_u, r_u, rep_u, soce, eh_p, r_p, rep_p, eh_n, r_n, rep_n,
               pr):
  eh_spec = pl.BlockSpec((C * LH, D), lambda i: (i, 0))
  r_spec = pl.BlockSpec((C, LH), lambda i: (i, 0))
  rep_spec = pl.BlockSpec((C, D), lambda i: (i, 0))
  soc_spec = pl.BlockSpec((C * LS, D), lambda i: (i, 0))
  pr_specs = jax.tree.map(_full, pr)
  f32 = jnp.float32
  return pl.pallas_call(
      _tc1_body,
      grid=(G,),
      in_specs=[eh_spec, r_spec, rep_spec, soc_spec,
                eh_spec, r_spec, rep_spec,
                eh_spec, r_spec, rep_spec, pr_specs],
      out_specs=[rep_spec, rep_spec, rep_spec],
      out_shape=[jax.ShapeDtypeStruct((B, D), f32)] * 3,
  )(eh_u, r_u, rep_u, soce, eh_p, r_p, rep_p, eh_n, r_n, rep_n, pr)


def _tc_stage2(xu, xi, xj, p2):
  x_spec = pl.BlockSpec((B, D), lambda: (0, 0))
  return pl.pallas_call(
      _tc2_body,
      in_specs=[x_spec, x_spec, x_spec, jax.tree.map(_full, p2)],
      out_specs=pl.BlockSpec((1, 1), lambda: (0, 0)),
      out_shape=jax.ShapeDtypeStruct((1, 1), jnp.float32),
  )(xu, xi, xj, p2)


def kernel(nodes_u, nodes_pos, nodes_neg, hist_u, hist_ur, hist_v, hist_vr,
           soc_adj, params):
  pr, p2 = _prep_params(params)
  (eh_u, eh_p, eh_n, soce, r_u, r_p, r_n, rp_u, rp_p, rp_n) = _sc_gather(
      nodes_u, nodes_pos, nodes_neg, hist_u, hist_ur, hist_v, hist_vr,
      soc_adj, params["u2e"], params["v2e"])
  xu, xi, xj = _tc_stage1(eh_u, r_u, rp_u, soce, eh_p, r_p, rp_p,
                          eh_n, r_n, rp_n, pr)
  return _tc_stage2(xu, xi, xj, p2)[0, 0]
